# baseline (speedup 1.0000x reference)
import functools

import jax
import jax.numpy as jnp
from jax import lax
from jax.experimental import pallas as pl
from jax.experimental.pallas import tpu as pltpu
from jax.experimental.pallas import tpu_sc as plsc

EMB = 32
BATCH = 16384
NC = 2
NS = 16
NW = NC * NS
B_PER_W = BATCH // NW


def _sc_gather_body(user_hbm, track_hbm, uemb_hbm, temb_hbm, u_out, t_out,
                    uidx_v, tidx_v, sem):
    wid = lax.axis_index("s") * NC + lax.axis_index("c")
    base = wid * B_PER_W
    pltpu.sync_copy(user_hbm.at[pl.ds(base, B_PER_W)], uidx_v)
    pltpu.sync_copy(track_hbm.at[pl.ds(base, B_PER_W)], tidx_v)
    def body(g, carry):
        row = g * 16
        uvec = uidx_v[pl.ds(row, 16)]
        tvec = tidx_v[pl.ds(row, 16)]
        for j in range(16):
            pltpu.async_copy(uemb_hbm.at[uvec[j]], u_out.at[base + row + j], sem)
            pltpu.async_copy(temb_hbm.at[tvec[j]], t_out.at[base + row + j], sem)
        return carry

    lax.fori_loop(0, B_PER_W // 16, body, 0)
    pltpu.make_async_copy(
        uemb_hbm.at[pl.ds(0, B_PER_W)], u_out.at[pl.ds(base, B_PER_W)], sem
    ).wait()
    pltpu.make_async_copy(
        temb_hbm.at[pl.ds(0, B_PER_W)], t_out.at[pl.ds(base, B_PER_W)], sem
    ).wait()


@functools.cache
def _sc_gather():
    return pl.kernel(
        _sc_gather_body,
        out_type=(
            jax.ShapeDtypeStruct((BATCH, EMB), jnp.float32),
            jax.ShapeDtypeStruct((BATCH, EMB), jnp.float32),
        ),
        mesh=plsc.VectorSubcoreMesh(core_axis_name="c", subcore_axis_name="s"),
        scratch_types=[
            pltpu.VMEM((B_PER_W,), jnp.int32),
            pltpu.VMEM((B_PER_W,), jnp.int32),
            pltpu.SemaphoreType.DMA,
        ],
    )


def _mlp_body(u_ref, t_ref, w1u_ref, w1t_ref, b1_ref, w2_ref, b2_ref, out_ref):
    h = jnp.dot(u_ref[...], w1u_ref[...], preferred_element_type=jnp.float32)
    h = h + jnp.dot(t_ref[...], w1t_ref[...], preferred_element_type=jnp.float32)
    h = jnp.maximum(h + b1_ref[...], 0.0)
    logit = jnp.sum(h * w2_ref[...], axis=1, keepdims=True) + b2_ref[...]
    out_ref[...] = 1.0 / (1.0 + jnp.exp(-logit))


def _mlp(u, t, w1u, w1t, b1, w2, b2, block_b=2048, interpret=False):
    grid = BATCH // block_b
    return pl.pallas_call(
        _mlp_body,
        grid=(grid,),
        in_specs=[
            pl.BlockSpec((block_b, EMB), lambda i: (i, 0)),
            pl.BlockSpec((block_b, EMB), lambda i: (i, 0)),
            pl.BlockSpec((EMB, 64), lambda i: (0, 0)),
            pl.BlockSpec((EMB, 64), lambda i: (0, 0)),
            pl.BlockSpec((1, 64), lambda i: (0, 0)),
            pl.BlockSpec((1, 64), lambda i: (0, 0)),
            pl.BlockSpec((1, 1), lambda i: (0, 0)),
        ],
        out_specs=pl.BlockSpec((block_b, 1), lambda i: (i, 0)),
        out_shape=jax.ShapeDtypeStruct((BATCH, 1), jnp.float32),
        interpret=interpret,
    )(u, t, w1u, w1t, b1, w2, b2)


def kernel(user, track, user_emb, track_emb, W1, b1, W2, b2):
    user = user.astype(jnp.int32)
    track = track.astype(jnp.int32)
    u, t = _sc_gather()(user, track, user_emb, track_emb)
    w1u = W1[:, :EMB].T
    w1t = W1[:, EMB:].T
    out = _mlp(u, t, w1u, w1t, b1.reshape(1, 64), W2, b2.reshape(1, 1))
    return out.reshape(BATCH)


# speedup vs baseline: 1.8047x; 1.8047x over previous
import functools

import jax
import jax.numpy as jnp
from jax import lax
from jax.experimental import pallas as pl
from jax.experimental.pallas import tpu as pltpu
from jax.experimental.pallas import tpu_sc as plsc

EMB = 32
BATCH = 16384
NC = 2
NS = 16
NW = NC * NS
B_PER_W = BATCH // NW


CHUNK = 256
N_CHUNKS = B_PER_W // CHUNK


def _sc_gather_body(user_hbm, track_hbm, uemb_hbm, temb_hbm, u_out, t_out,
                    uidx_v, tidx_v, ubuf, tbuf, sem_u, sem_t):
    wid = lax.axis_index("s") * NC + lax.axis_index("c")
    base = wid * B_PER_W
    pltpu.sync_copy(user_hbm.at[pl.ds(base, B_PER_W)], uidx_v)
    pltpu.sync_copy(track_hbm.at[pl.ds(base, B_PER_W)], tidx_v)

    for c in range(N_CHUNKS):
        row0 = c * CHUNK

        def group(g, carry):
            row = row0 + g * 16
            uvec = uidx_v[pl.ds(row, 16)]
            tvec = tidx_v[pl.ds(row, 16)]
            for j in range(16):
                dst = g * 16 + j
                pltpu.async_copy(uemb_hbm.at[uvec[j]], ubuf.at[dst], sem_u)
                pltpu.async_copy(temb_hbm.at[tvec[j]], tbuf.at[dst], sem_t)
            return carry

        lax.fori_loop(0, CHUNK // 16, group, 0)
        pltpu.make_async_copy(uemb_hbm.at[pl.ds(0, CHUNK)], ubuf, sem_u).wait()
        pltpu.make_async_copy(temb_hbm.at[pl.ds(0, CHUNK)], tbuf, sem_t).wait()
        pltpu.sync_copy(ubuf, u_out.at[pl.ds(base + row0, CHUNK)])
        pltpu.sync_copy(tbuf, t_out.at[pl.ds(base + row0, CHUNK)])


@functools.cache
def _sc_gather():
    return pl.kernel(
        _sc_gather_body,
        out_type=(
            jax.ShapeDtypeStruct((BATCH, EMB), jnp.float32),
            jax.ShapeDtypeStruct((BATCH, EMB), jnp.float32),
        ),
        mesh=plsc.VectorSubcoreMesh(core_axis_name="c", subcore_axis_name="s"),
        scratch_types=[
            pltpu.VMEM((B_PER_W,), jnp.int32),
            pltpu.VMEM((B_PER_W,), jnp.int32),
            pltpu.VMEM((CHUNK, EMB), jnp.float32),
            pltpu.VMEM((CHUNK, EMB), jnp.float32),
            pltpu.SemaphoreType.DMA,
            pltpu.SemaphoreType.DMA,
        ],
    )


def _mlp_body(u_ref, t_ref, w1u_ref, w1t_ref, b1_ref, w2_ref, b2_ref, out_ref):
    h = jnp.dot(u_ref[...], w1u_ref[...], preferred_element_type=jnp.float32)
    h = h + jnp.dot(t_ref[...], w1t_ref[...], preferred_element_type=jnp.float32)
    h = jnp.maximum(h + b1_ref[...], 0.0)
    logit = jnp.sum(h * w2_ref[...], axis=1, keepdims=True) + b2_ref[...]
    out_ref[...] = 1.0 / (1.0 + jnp.exp(-logit))


def _mlp(u, t, w1u, w1t, b1, w2, b2, block_b=2048, interpret=False):
    grid = BATCH // block_b
    return pl.pallas_call(
        _mlp_body,
        grid=(grid,),
        in_specs=[
            pl.BlockSpec((block_b, EMB), lambda i: (i, 0)),
            pl.BlockSpec((block_b, EMB), lambda i: (i, 0)),
            pl.BlockSpec((EMB, 64), lambda i: (0, 0)),
            pl.BlockSpec((EMB, 64), lambda i: (0, 0)),
            pl.BlockSpec((1, 64), lambda i: (0, 0)),
            pl.BlockSpec((1, 64), lambda i: (0, 0)),
            pl.BlockSpec((1, 1), lambda i: (0, 0)),
        ],
        out_specs=pl.BlockSpec((block_b, 1), lambda i: (i, 0)),
        out_shape=jax.ShapeDtypeStruct((BATCH, 1), jnp.float32),
        interpret=interpret,
    )(u, t, w1u, w1t, b1, w2, b2)


def kernel(user, track, user_emb, track_emb, W1, b1, W2, b2):
    user = user.astype(jnp.int32)
    track = track.astype(jnp.int32)
    u, t = _sc_gather()(user, track, user_emb, track_emb)
    w1u = W1[:, :EMB].T
    w1t = W1[:, EMB:].T
    out = _mlp(u, t, w1u, w1t, b1.reshape(1, 64), W2, b2.reshape(1, 1))
    return out.reshape(BATCH)
